# baseline (speedup 1.0000x reference)
import functools

import jax
import jax.numpy as jnp
from jax import lax
from jax.experimental import pallas as pl
from jax.experimental.pallas import tpu as pltpu
from jax.experimental.pallas import tpu_sc as plsc

_B, _T, _I = 2, 2048, 1024
_E, _K, _J = 8, 512, 512

_INFO = plsc.get_sparse_core_info()
_NC, _NS = _INFO.num_cores, _INFO.num_subcores
_NW = _NC * _NS

_ES = 2
_WPE = 16 // _ES
_CROWS = _B * _ES * _K
_RPW = _CROWS // _NW
_CHUNK = 32
_NCHUNK = _RPW // _CHUNK


def _sc_gather_slab(x_flat, ind2, c):
    mesh = plsc.VectorSubcoreMesh(core_axis_name="c", subcore_axis_name="s")

    @functools.partial(
        pl.kernel,
        mesh=mesh,
        out_type=jax.ShapeDtypeStruct((_CROWS, _I), jnp.float32),
        scratch_types=[
            pltpu.VMEM((_NCHUNK, _CHUNK), jnp.int32),
            pltpu.VMEM((_CHUNK, _I), jnp.float32),
            pltpu.VMEM((_CHUNK, _I), jnp.float32),
            pltpu.SemaphoreType.DMA,
            pltpu.SemaphoreType.DMA,
            pltpu.SemaphoreType.DMA,
            pltpu.SemaphoreType.DMA,
        ],
    )
    def gather_kernel(x_hbm, ind_hbm, out_hbm, idx_v, rows0, rows1, gs0, gs1,
                      os0, os1):
        wid = lax.axis_index("s") * _NC + lax.axis_index("c")
        b = wid // 16
        u = wid % 16
        boff = b * _T
        ind_blk = ((b * _E + _ES * c + u // _WPE) * _K
                   + (u % _WPE) * _RPW) // _RPW
        base = wid * _RPW

        pltpu.sync_copy(ind_hbm.at[ind_blk], idx_v)
        for r in range(_NCHUNK):
            for h in range(_CHUNK // 16):
                sl = pl.ds(h * 16, 16)
                idx_v[r, sl] = idx_v[r, sl] + boff

        rows = (rows0, rows1)
        gs = (gs0, gs1)
        os = (os0, os1)

        def start_gather(s):
            return pltpu.async_copy(x_hbm.at[idx_v.at[s]], rows[s % 2],
                                    gs[s % 2])

        def start_out(s):
            return pltpu.async_copy(
                rows[s % 2], out_hbm.at[pl.ds(base + s * _CHUNK, _CHUNK)],
                os[s % 2])

        g_h = [None] * _NCHUNK
        o_h = [None] * _NCHUNK
        g_h[0] = start_gather(0)
        g_h[1] = start_gather(1)
        for s in range(_NCHUNK):
            g_h[s].wait()
            o_h[s] = start_out(s)
            if s + 2 < _NCHUNK:
                o_h[s].wait()
                g_h[s + 2] = start_gather(s + 2)
        o_h[_NCHUNK - 2].wait()
        o_h[_NCHUNK - 1].wait()

    return gather_kernel(x_flat, ind2)


def _tc_matmul_slab(xg, w, c, y_prev=None):
    def mm_kernel(*refs):
        x_ref, w_ref, o_ref = refs[-3:]
        for bi in range(_B):
            o_ref[bi, 0] = jnp.dot(x_ref[bi, 0], w_ref[0],
                                   preferred_element_type=jnp.float32)

    out_shape = jax.ShapeDtypeStruct((_B, _E, _K, _J), jnp.float32)
    mm_specs = [
        pl.BlockSpec((_B, 1, _K, _I), lambda e: (0, e, 0, 0)),
        pl.BlockSpec((1, _I, _J), lambda e: (_ES * c + e, 0, 0)),
    ]
    out_spec = pl.BlockSpec((_B, 1, _K, _J),
                            lambda e: (0, _ES * c + e, 0, 0))
    if y_prev is None:
        return pl.pallas_call(
            mm_kernel, grid=(_ES,), in_specs=mm_specs, out_specs=out_spec,
            out_shape=out_shape)(xg, w)
    return pl.pallas_call(
        mm_kernel, grid=(_ES,),
        in_specs=[pl.BlockSpec(memory_space=pl.ANY)] + mm_specs,
        out_specs=out_spec, out_shape=out_shape,
        input_output_aliases={0: 0})(y_prev, xg, w)


def kernel(X, ind, W):
    x_flat = X.reshape(_B * _T, _I)
    ind2 = ind.reshape((_B * _E * _K) // _RPW, _NCHUNK, _CHUNK)
    nslabs = _E // _ES
    xgs = [_sc_gather_slab(x_flat, ind2, c) for c in range(nslabs)]
    y = None
    for c in range(nslabs):
        y = _tc_matmul_slab(xgs[c].reshape(_B, _ES, _K, _I), W, c, y_prev=y)
    return y


# speedup vs baseline: 1.0074x; 1.0074x over previous
import functools

import jax
import jax.numpy as jnp
from jax import lax
from jax.experimental import pallas as pl
from jax.experimental.pallas import tpu as pltpu
from jax.experimental.pallas import tpu_sc as plsc

_B, _T, _I = 2, 2048, 1024
_E, _K, _J = 8, 512, 512

_INFO = plsc.get_sparse_core_info()
_NC, _NS = _INFO.num_cores, _INFO.num_subcores
_NW = _NC * _NS

_CHUNK = 32
_SLABS = ((0, 2), (2, 4), (6, 2))


def _sc_gather_slab(x_flat, ind3, e0, es):
    crows = _B * es * _K
    rpw = crows // _NW
    nch = rpw // _CHUNK
    nblk = rpw // 64
    wpe = 16 // es
    mesh = plsc.VectorSubcoreMesh(core_axis_name="c", subcore_axis_name="s")

    @functools.partial(
        pl.kernel,
        mesh=mesh,
        out_type=jax.ShapeDtypeStruct((crows, _I), jnp.float32),
        scratch_types=[
            pltpu.VMEM((nblk, 2, _CHUNK), jnp.int32),
            pltpu.VMEM((_CHUNK, _I), jnp.float32),
            pltpu.VMEM((_CHUNK, _I), jnp.float32),
            pltpu.SemaphoreType.DMA,
            pltpu.SemaphoreType.DMA,
            pltpu.SemaphoreType.DMA,
            pltpu.SemaphoreType.DMA,
        ],
    )
    def gather_kernel(x_hbm, ind_hbm, out_hbm, idx_v, rows0, rows1, gs0, gs1,
                      os0, os1):
        wid = lax.axis_index("s") * _NC + lax.axis_index("c")
        b = wid // 16
        u = wid % 16
        boff = b * _T
        blk0 = ((b * _E + e0 + u // wpe) * _K + (u % wpe) * rpw) // 64
        base = wid * rpw

        pltpu.sync_copy(ind_hbm.at[pl.ds(blk0, nblk)], idx_v)
        for r in range(nblk):
            for j in range(2):
                for h in range(_CHUNK // 16):
                    sl = pl.ds(h * 16, 16)
                    idx_v[r, j, sl] = idx_v[r, j, sl] + boff

        rows = (rows0, rows1)
        gs = (gs0, gs1)
        os = (os0, os1)

        def start_gather(s):
            return pltpu.async_copy(x_hbm.at[idx_v.at[s // 2, s % 2]],
                                    rows[s % 2], gs[s % 2])

        def start_out(s):
            return pltpu.async_copy(
                rows[s % 2], out_hbm.at[pl.ds(base + s * _CHUNK, _CHUNK)],
                os[s % 2])

        g_h = [None] * nch
        o_h = [None] * nch
        g_h[0] = start_gather(0)
        g_h[1] = start_gather(1)
        for s in range(nch):
            g_h[s].wait()
            o_h[s] = start_out(s)
            if s + 2 < nch:
                o_h[s].wait()
                g_h[s + 2] = start_gather(s + 2)
        o_h[nch - 2].wait()
        o_h[nch - 1].wait()

    return gather_kernel(x_flat, ind3)


def _tc_matmul_slab(xg, w, e0, es, y_prev=None):
    def mm_kernel(*refs):
        x_ref, w_ref, o_ref = refs[-3:]
        for bi in range(_B):
            o_ref[bi, 0] = jnp.dot(x_ref[bi, 0], w_ref[0],
                                   preferred_element_type=jnp.float32)

    out_shape = jax.ShapeDtypeStruct((_B, _E, _K, _J), jnp.float32)
    mm_specs = [
        pl.BlockSpec((_B, 1, _K, _I), lambda e: (0, e, 0, 0)),
        pl.BlockSpec((1, _I, _J), lambda e: (e0 + e, 0, 0)),
    ]
    out_spec = pl.BlockSpec((_B, 1, _K, _J), lambda e: (0, e0 + e, 0, 0))
    if y_prev is None:
        return pl.pallas_call(
            mm_kernel, grid=(es,), in_specs=mm_specs, out_specs=out_spec,
            out_shape=out_shape)(xg, w)
    return pl.pallas_call(
        mm_kernel, grid=(es,),
        in_specs=[pl.BlockSpec(memory_space=pl.ANY)] + mm_specs,
        out_specs=out_spec, out_shape=out_shape,
        input_output_aliases={0: 0})(y_prev, xg, w)


def kernel(X, ind, W):
    x_flat = X.reshape(_B * _T, _I)
    ind3 = ind.reshape((_B * _E * _K) // 64, 2, _CHUNK)
    xgs = [_sc_gather_slab(x_flat, ind3, e0, es) for e0, es in _SLABS]
    y = None
    for (e0, es), xg in zip(_SLABS, xgs):
        y = _tc_matmul_slab(xg.reshape(_B, es, _K, _I), W, e0, es, y_prev=y)
    return y
